# baseline (speedup 1.0000x reference)
import functools

import jax
import jax.numpy as jnp
from jax.experimental import pallas as pl
from jax.experimental.pallas import tpu as pltpu


def _round_up(x, m):
    return (x + m - 1) // m * m


def _rowsum_tree(a_bf, tk):
    chunks = [a_bf[:, c * 128:(c + 1) * 128] for c in range(tk // 128)]
    while len(chunks) > 1:
        chunks = [chunks[i] + chunks[i + 1] for i in range(0, len(chunks), 2)]
    return chunks[0].astype(jnp.float32)


def _gcn_kernel(at_ref, ab_ref, x_ref, w_ref, b_ref, o_ref, acc_ref, rs_ref,
                *, tk, nk, nh):
    k = pl.program_id(0)

    @pl.when(k == 0)
    def _init():
        acc_ref[...] = jnp.zeros_like(acc_ref)
        rs_ref[...] = jnp.zeros_like(rs_ref)

    at = at_ref[...]
    ab = ab_ref[...]
    at_bf = at.astype(jnp.bfloat16)
    ab_bf = ab.astype(jnp.bfloat16)

    colsum = (jnp.sum(at, axis=0, keepdims=True) +
              jnp.sum(ab, axis=0, keepdims=True))
    s = jnp.transpose(jax.lax.rsqrt(jnp.maximum(colsum, 1.0)))

    rs_ref[0:nh, :] += _rowsum_tree(at_bf, tk)
    rs_ref[nh:2 * nh, :] += _rowsum_tree(ab_bf, tk)

    xs = (x_ref[...] * s).astype(jnp.bfloat16)
    acc_ref[0:nh, :] += jnp.dot(at_bf, xs, preferred_element_type=jnp.float32)
    acc_ref[nh:2 * nh, :] += jnp.dot(ab_bf, xs,
                                     preferred_element_type=jnp.float32)

    @pl.when(k == nk - 1)
    def _finalize():
        rowsum = jnp.sum(rs_ref[...], axis=1, keepdims=True)
        norm_dst = jax.lax.rsqrt(jnp.maximum(rowsum, 1.0))
        out = jnp.dot(acc_ref[...], w_ref[...],
                      preferred_element_type=jnp.float32)
        o_ref[...] = jnp.maximum(out * norm_dst + b_ref[...], 0.0)


@functools.partial(jax.jit, static_argnames=("tile_k",))
def _gcn_forward(adj, feats, weight, bias, *, tile_k=1024):
    n, f_in = feats.shape
    f_out = weight.shape[1]

    n_pad = _round_up(n, 128)
    f_in_p = _round_up(f_in, 128)
    f_out_p = _round_up(f_out, 128)

    tk = min(tile_k, n_pad)
    while n_pad % tk:
        tk -= 128
    nk = n_pad // tk

    adj = adj.astype(jnp.float32)
    feats = feats.astype(jnp.float32)
    if n_pad != n:
        adj = jnp.pad(adj, ((0, n_pad - n), (0, n_pad - n)))
        feats = jnp.pad(feats, ((0, n_pad - n), (0, 0)))
    if f_in_p != f_in:
        feats = jnp.pad(feats, ((0, 0), (0, f_in_p - f_in)))

    if (f_in_p, f_out_p) == (f_in, f_out):
        w_p = weight.astype(jnp.float32)
        b_p = bias.astype(jnp.float32).reshape(1, f_out)
    else:
        w_p = jnp.zeros((f_in_p, f_out_p), jnp.float32)
        w_p = w_p.at[:f_in, :f_out].set(weight.astype(jnp.float32))
        b_p = jnp.zeros((1, f_out_p), jnp.float32)
        b_p = b_p.at[:, :f_out].set(bias.astype(jnp.float32))

    nh = n_pad // 2
    out_p = pl.pallas_call(
        functools.partial(_gcn_kernel, tk=tk, nk=nk, nh=nh),
        out_shape=jax.ShapeDtypeStruct((n_pad, f_out_p), jnp.float32),
        grid_spec=pltpu.PrefetchScalarGridSpec(
            num_scalar_prefetch=0,
            grid=(nk,),
            in_specs=[
                pl.BlockSpec((nh, tk), lambda k: (0, k)),
                pl.BlockSpec((nh, tk), lambda k: (1, k)),
                pl.BlockSpec((tk, f_in_p), lambda k: (k, 0)),
                pl.BlockSpec((f_in_p, f_out_p), lambda k: (0, 0)),
                pl.BlockSpec((1, f_out_p), lambda k: (0, 0)),
            ],
            out_specs=pl.BlockSpec((n_pad, f_out_p), lambda k: (0, 0)),
            scratch_shapes=[
                pltpu.VMEM((n_pad, f_in_p), jnp.float32),
                pltpu.VMEM((n_pad, 128), jnp.float32),
            ],
        ),
        compiler_params=pltpu.CompilerParams(
            dimension_semantics=("arbitrary",),
            vmem_limit_bytes=58 * 1024 * 1024,
        ),
    )(adj, adj, feats, w_p, b_p)

    return out_p[:n, :f_out]


def kernel(adj, feats, weight, bias):
    return _gcn_forward(adj, feats, weight, bias)


# speedup vs baseline: 1.0850x; 1.0850x over previous
import functools

import jax
import jax.numpy as jnp
from jax.experimental import pallas as pl
from jax.experimental.pallas import tpu as pltpu


def _round_up(x, m):
    return (x + m - 1) // m * m


def _gcn_kernel(a_ref, x_ref, w_ref, b_ref, o_ref, acc_ref, rs_ref,
                *, tk, nk):
    k = pl.program_id(0)

    a = a_ref[...]
    a_bf = a.astype(jnp.bfloat16)

    colsum = jnp.sum(a, axis=0, keepdims=True)
    s = jnp.transpose(jax.lax.rsqrt(jnp.maximum(colsum, 1.0)))

    chunks = [a_bf[:, c * 128:(c + 1) * 128] for c in range(tk // 128)]
    while len(chunks) > 1:
        chunks = [chunks[i] + chunks[i + 1] for i in range(0, len(chunks), 2)]
    rsum = chunks[0].astype(jnp.float32)

    xs = (x_ref[...] * s).astype(jnp.bfloat16)
    contrib = jnp.dot(a_bf, xs, preferred_element_type=jnp.float32)

    @pl.when(k == 0)
    def _first():
        acc_ref[...] = contrib
        rs_ref[...] = rsum

    @pl.when(k > 0)
    def _rest():
        acc_ref[...] += contrib
        rs_ref[...] += rsum

    @pl.when(k == nk - 1)
    def _finalize():
        rowsum = jnp.sum(rs_ref[...], axis=1, keepdims=True)
        norm_dst = jax.lax.rsqrt(jnp.maximum(rowsum, 1.0))
        out = jnp.dot(acc_ref[...], w_ref[...],
                      preferred_element_type=jnp.float32)
        o_ref[...] = jnp.maximum(out * norm_dst + b_ref[...], 0.0)


@functools.partial(jax.jit, static_argnames=("tile_k",))
def _gcn_forward(adj, feats, weight, bias, *, tile_k=1024):
    n, f_in = feats.shape
    f_out = weight.shape[1]

    n_pad = _round_up(n, 128)
    f_in_p = _round_up(f_in, 128)
    f_out_p = _round_up(f_out, 128)

    tk = min(tile_k, n_pad)
    while n_pad % tk:
        tk -= 128
    nk = n_pad // tk

    adj = adj.astype(jnp.float32)
    feats = feats.astype(jnp.float32)
    if n_pad != n:
        adj = jnp.pad(adj, ((0, n_pad - n), (0, n_pad - n)))
        feats = jnp.pad(feats, ((0, n_pad - n), (0, 0)))
    if f_in_p != f_in:
        feats = jnp.pad(feats, ((0, 0), (0, f_in_p - f_in)))

    if (f_in_p, f_out_p) == (f_in, f_out):
        w_p = weight.astype(jnp.float32)
        b_p = bias.astype(jnp.float32).reshape(1, f_out)
    else:
        w_p = jnp.zeros((f_in_p, f_out_p), jnp.float32)
        w_p = w_p.at[:f_in, :f_out].set(weight.astype(jnp.float32))
        b_p = jnp.zeros((1, f_out_p), jnp.float32)
        b_p = b_p.at[:, :f_out].set(bias.astype(jnp.float32))

    out_p = pl.pallas_call(
        functools.partial(_gcn_kernel, tk=tk, nk=nk),
        out_shape=jax.ShapeDtypeStruct((n_pad, f_out_p), jnp.float32),
        grid_spec=pltpu.PrefetchScalarGridSpec(
            num_scalar_prefetch=0,
            grid=(nk,),
            in_specs=[
                pl.BlockSpec((n_pad, tk), lambda k: (0, k)),
                pl.BlockSpec((tk, f_in_p), lambda k: (k, 0)),
                pl.BlockSpec((f_in_p, f_out_p), lambda k: (0, 0)),
                pl.BlockSpec((1, f_out_p), lambda k: (0, 0)),
            ],
            out_specs=pl.BlockSpec((n_pad, f_out_p), lambda k: (0, 0)),
            scratch_shapes=[
                pltpu.VMEM((n_pad, f_in_p), jnp.float32),
                pltpu.VMEM((n_pad, 128), jnp.float32),
            ],
        ),
        compiler_params=pltpu.CompilerParams(
            dimension_semantics=("arbitrary",),
            vmem_limit_bytes=58 * 1024 * 1024,
        ),
    )(adj, feats, w_p, b_p)

    return out_p[:n, :f_out]


def kernel(adj, feats, weight, bias):
    return _gcn_forward(adj, feats, weight, bias)
